# baseline (speedup 1.0000x reference)
import functools

import jax
import jax.numpy as jnp
from jax import lax
from jax.experimental import pallas as pl
from jax.experimental.pallas import tpu as pltpu
from jax.experimental.pallas import tpu_sc as plsc

_L = 16
_NB = 3
_G = 2


def _make_sc_kernel(B, S, D):
    info = plsc.get_sparse_core_info()
    NC, NS = info.num_cores, info.num_subcores
    NW = NC * NS
    SW = S // NW
    R = 16
    n_chunks = SW // R
    NG = B // _G
    n_jobs = n_chunks * NG

    mesh = plsc.VectorSubcoreMesh(core_axis_name="c", subcore_axis_name="s")

    @functools.partial(
        pl.kernel,
        out_type=jax.ShapeDtypeStruct((B, S, D), jnp.float32),
        mesh=mesh,
        scratch_types=[
            pltpu.VMEM((R, D), jnp.float32),
            pltpu.VMEM((_NB, _G, R, D), jnp.float32),
            pltpu.SemaphoreType.DMA,
            pltpu.SemaphoreType.DMA,
            pltpu.SemaphoreType.DMA,
        ],
        compiler_params=pltpu.CompilerParams(use_tc_tiling_on_sc=True),
    )
    def body(x_hbm, pos_hbm, out_hbm, pbuf, xbufs, in_sem, out_sem, p_sem):
        wid = lax.axis_index("s") * NC + lax.axis_index("c")
        base = wid * SW

        def in_cp(j, k):
            c, g = divmod(j, NG)
            return pltpu.make_async_copy(
                x_hbm.at[pl.ds(g * _G, _G), pl.ds(base + c * R, R), :],
                xbufs.at[k], in_sem)

        def out_cp(j, k):
            c, g = divmod(j, NG)
            return pltpu.make_async_copy(
                xbufs.at[k],
                out_hbm.at[pl.ds(g * _G, _G), pl.ds(base + c * R, R), :],
                out_sem)

        def p_cp(c):
            return pltpu.make_async_copy(
                pos_hbm.at[pl.ds(base + c * R, R), :], pbuf, p_sem)

        p_cp(0).start()
        in_cp(0, 0).start()
        in_cp(1, 1).start()
        for j in range(n_jobs):
            k = j % _NB
            c, g = divmod(j, NG)
            if g == 0:
                p_cp(c).wait()
            in_cp(j, k).wait()
            if j + 2 < n_jobs:
                if j + 2 - _NB >= 0:
                    out_cp(j + 2 - _NB, (j + 2) % _NB).wait()
                in_cp(j + 2, (j + 2) % _NB).start()

            xb = xbufs.at[k]

            @plsc.parallel_loop(0, (R * D) // _L, unroll=4)
            def _add(i):
                r = i >> 6
                sl = pl.ds((i & (D // _L - 1)) * _L, _L)
                pv = pbuf[r, sl]
                for b in range(_G):
                    xb[b, r, sl] = xb[b, r, sl] + pv

            if g == NG - 1 and c + 1 < n_chunks:
                p_cp(c + 1).start()
            out_cp(j, k).start()
        for j in range(max(0, n_jobs - _NB), n_jobs):
            out_cp(j, j % _NB).wait()

    return body


def kernel(x, pos_table):
    B, S, D = x.shape
    sc = _make_sc_kernel(B, S, D)
    return sc(x, pos_table)


# speedup vs baseline: 1.0713x; 1.0713x over previous
import functools

import jax
import jax.numpy as jnp
from jax import lax
from jax.experimental import pallas as pl
from jax.experimental.pallas import tpu as pltpu
from jax.experimental.pallas import tpu_sc as plsc

_L = 16
_NB = 3


def _make_sc_kernel(B, S, D):
    info = plsc.get_sparse_core_info()
    NC, NS = info.num_cores, info.num_subcores
    NW = NC * NS
    SW = S // NW
    R = 8
    n_chunks = SW // R

    mesh = plsc.VectorSubcoreMesh(core_axis_name="c", subcore_axis_name="s")

    @functools.partial(
        pl.kernel,
        out_type=jax.ShapeDtypeStruct((B, S, D), jnp.float32),
        mesh=mesh,
        scratch_types=[
            pltpu.VMEM((2, R, D), jnp.float32),
            pltpu.VMEM((_NB, B, R, D), jnp.float32),
            pltpu.SemaphoreType.DMA,
            pltpu.SemaphoreType.DMA,
            pltpu.SemaphoreType.DMA,
        ],
        compiler_params=pltpu.CompilerParams(use_tc_tiling_on_sc=True),
    )
    def body(x_hbm, pos_hbm, out_hbm, pbuf, xbufs, in_sem, out_sem, p_sem):
        wid = lax.axis_index("s") * NC + lax.axis_index("c")
        base = wid * SW

        def in_cp(c, k):
            return pltpu.make_async_copy(
                x_hbm.at[:, pl.ds(base + c * R, R), :], xbufs.at[k], in_sem)

        def out_cp(c, k):
            return pltpu.make_async_copy(
                xbufs.at[k], out_hbm.at[:, pl.ds(base + c * R, R), :], out_sem)

        def p_cp(c, k):
            return pltpu.make_async_copy(
                pos_hbm.at[pl.ds(base + c * R, R), :], pbuf.at[k], p_sem)

        p_cp(0, 0).start()
        in_cp(0, 0).start()
        in_cp(1, 1).start()
        for c in range(n_chunks):
            k = c % _NB
            p_cp(c, c % 2).wait()
            if c + 1 < n_chunks:
                p_cp(c + 1, (c + 1) % 2).start()
            in_cp(c, k).wait()
            if c + 2 < n_chunks:
                if c + 2 - _NB >= 0:
                    out_cp(c + 2 - _NB, (c + 2) % _NB).wait()
                in_cp(c + 2, (c + 2) % _NB).start()

            xb = xbufs.at[k]
            pb = pbuf.at[c % 2]

            @plsc.parallel_loop(0, (R * D) // _L, unroll=4)
            def _add(i):
                r = i >> 6
                sl = pl.ds((i & (D // _L - 1)) * _L, _L)
                pv = pb[r, sl]
                for b in range(B):
                    xb[b, r, sl] = xb[b, r, sl] + pv

            out_cp(c, k).start()
        for c in range(max(0, n_chunks - _NB), n_chunks):
            out_cp(c, c % _NB).wait()

    return body


def kernel(x, pos_table):
    B, S, D = x.shape
    sc = _make_sc_kernel(B, S, D)
    return sc(x, pos_table)
